# Initial kernel scaffold; baseline (speedup 1.0000x reference)
import jax
import jax.numpy as jnp
from jax.experimental import pallas as pl


def kernel(nodes_fea, edges, edges_type, W_in, b_in, W_rel, b_rel, W_loop, W1, b1, W2, b2):
    raise NotImplementedError("write your pallas kernel here")


# speedup vs baseline: 18.9793x; 18.9793x over previous
import functools

import jax
import jax.numpy as jnp
from jax import lax
from jax.experimental import pallas as pl
from jax.experimental.pallas import tpu as pltpu
from jax.experimental.pallas import tpu_sc as plsc

N = 10000
E = 320000
F = 128
H = 128
R = 4
L = 10

NC = 2
NS = 16
NW = NC * NS
CK = 128
NCH = 79
EP = NW * NCH * CK
NA = NCH * CK


def _sc_agg_body(xr_hbm, gidx_hbm, dst_hbm, out_hbm, gidx_v, dst_v, rows_v, acc_sh, sem):
    c = lax.axis_index("c")
    s = lax.axis_index("s")
    w = c * NS + s

    pltpu.sync_copy(gidx_hbm.at[w], gidx_v)
    pltpu.sync_copy(dst_hbm.at[w], dst_v)

    zeros16 = jnp.zeros((16,), jnp.float32)

    def zero_body(i, _):
        j = i // (H // 16)
        k = i % (H // 16)
        rows_v[j, pl.ds(k * 16, 16)] = zeros16
        return 0

    lax.fori_loop(0, CK * (H // 16), zero_body, 0)
    for q in range(5):
        k = s + NS * q
        if q == 4:
            @pl.when(k < NCH)
            def _():
                r0 = pl.multiple_of(k * CK, CK)
                pltpu.sync_copy(rows_v, acc_sh.at[pl.ds(r0, CK)])
        else:
            r0 = pl.multiple_of(k * CK, CK)
            pltpu.sync_copy(rows_v, acc_sh.at[pl.ds(r0, CK)])

    plsc.subcore_barrier()

    def chunk_body(j, _):
        pltpu.async_copy(xr_hbm.at[gidx_v.at[j]], rows_v, sem).wait()
        pltpu.sync_copy(rows_v, acc_sh.at[dst_v.at[j]], add=True)
        return 0

    lax.fori_loop(0, NCH, chunk_body, 0)

    plsc.subcore_barrier()

    for q in range(5):
        k = s + NS * q

        @pl.when(k < N // CK)
        def _():
            r0 = pl.multiple_of(k * CK, CK)
            pltpu.sync_copy(acc_sh.at[pl.ds(r0, CK)], rows_v)
            pltpu.sync_copy(rows_v, out_hbm.at[c, pl.ds(r0, CK)])

        @pl.when(k == N // CK)
        def _():
            r0 = pl.multiple_of(k * CK, CK)
            tail = N - (N // CK) * CK
            pltpu.sync_copy(acc_sh.at[pl.ds(r0, tail)], rows_v.at[pl.ds(0, tail)])
            pltpu.sync_copy(rows_v.at[pl.ds(0, tail)], out_hbm.at[c, pl.ds(r0, tail)])


_sc_agg = functools.partial(
    pl.kernel,
    mesh=plsc.VectorSubcoreMesh(core_axis_name="c", subcore_axis_name="s"),
    out_type=jax.ShapeDtypeStruct((NC, N, H), jnp.float32),
    scratch_types=[
        pltpu.VMEM((NCH, CK), jnp.int32),
        pltpu.VMEM((NCH, CK), jnp.int32),
        pltpu.VMEM((CK, H), jnp.float32),
        pltpu.VMEM_SHARED((NA, H), jnp.float32),
        pltpu.SemaphoreType.DMA,
    ],
)(_sc_agg_body)


BN = 1000


def _mm(a, b):
    return jnp.dot(a, b, preferred_element_type=jnp.float32)


def _initpre_body(nf_ref, win_ref, bin_ref, wrel_ref, x_ref, xr_ref):
    x = jnp.tanh(_mm(nf_ref[...], win_ref[...]) + bin_ref[...])
    x_ref[...] = x
    for r in range(R):
        xr_ref[r] = _mm(x, wrel_ref[r])


def _initpre(nodes_fea, W_in, b_in, W_rel0):
    return pl.pallas_call(
        _initpre_body,
        grid=(N // BN,),
        in_specs=[
            pl.BlockSpec((BN, F), lambda i: (i, 0)),
            pl.BlockSpec((F, H), lambda i: (0, 0)),
            pl.BlockSpec((1, H), lambda i: (0, 0)),
            pl.BlockSpec((R, H, H), lambda i: (0, 0, 0)),
        ],
        out_specs=[
            pl.BlockSpec((BN, H), lambda i: (i, 0)),
            pl.BlockSpec((R, BN, H), lambda i: (0, i, 0)),
        ],
        out_shape=[
            jax.ShapeDtypeStruct((N, H), jnp.float32),
            jax.ShapeDtypeStruct((R, N, H), jnp.float32),
        ],
    )(nodes_fea, W_in, b_in, W_rel0)


def _update_core(acc_ref, x_ref, wloop_ref, brel_ref, w1_ref, b1_ref, w2_ref, b2_ref):
    x = x_ref[...]
    msg = acc_ref[0] + acc_ref[1] + _mm(x, wloop_ref[...]) + brel_ref[...]
    mid = jnp.tanh(_mm(x, w1_ref[:H]) + _mm(msg, w1_ref[H:]) + b1_ref[...])
    xn = jnp.tanh(_mm(x, w2_ref[:H]) + _mm(mid, w2_ref[H:]) + b2_ref[...])
    return xn


def _postpre_body(acc_ref, x_ref, wloop_ref, brel_ref, w1_ref, b1_ref, w2_ref,
                  b2_ref, wrel_ref, xn_ref, xr_ref):
    xn = _update_core(acc_ref, x_ref, wloop_ref, brel_ref, w1_ref, b1_ref,
                      w2_ref, b2_ref)
    xn_ref[...] = xn
    for r in range(R):
        xr_ref[r] = _mm(xn, wrel_ref[r])


def _post_body(acc_ref, x_ref, wloop_ref, brel_ref, w1_ref, b1_ref, w2_ref,
               b2_ref, xn_ref):
    xn_ref[...] = _update_core(acc_ref, x_ref, wloop_ref, brel_ref, w1_ref,
                               b1_ref, w2_ref, b2_ref)


_UPDATE_IN_SPECS = [
    pl.BlockSpec((NC, BN, H), lambda i: (0, i, 0)),
    pl.BlockSpec((BN, H), lambda i: (i, 0)),
    pl.BlockSpec((H, H), lambda i: (0, 0)),
    pl.BlockSpec((1, H), lambda i: (0, 0)),
    pl.BlockSpec((2 * H, 2 * H), lambda i: (0, 0)),
    pl.BlockSpec((1, 2 * H), lambda i: (0, 0)),
    pl.BlockSpec((3 * H, H), lambda i: (0, 0)),
    pl.BlockSpec((1, H), lambda i: (0, 0)),
]


def _postpre(acc2, x, W_loop_l, b_rel_l, W1_l, b1_l, W2_l, b2_l, W_rel_n):
    return pl.pallas_call(
        _postpre_body,
        grid=(N // BN,),
        in_specs=_UPDATE_IN_SPECS + [pl.BlockSpec((R, H, H), lambda i: (0, 0, 0))],
        out_specs=[
            pl.BlockSpec((BN, H), lambda i: (i, 0)),
            pl.BlockSpec((R, BN, H), lambda i: (0, i, 0)),
        ],
        out_shape=[
            jax.ShapeDtypeStruct((N, H), jnp.float32),
            jax.ShapeDtypeStruct((R, N, H), jnp.float32),
        ],
    )(acc2, x, W_loop_l, b_rel_l, W1_l, b1_l, W2_l, b2_l, W_rel_n)


def _post(acc2, x, W_loop_l, b_rel_l, W1_l, b1_l, W2_l, b2_l):
    return pl.pallas_call(
        _post_body,
        grid=(N // BN,),
        in_specs=_UPDATE_IN_SPECS,
        out_specs=pl.BlockSpec((BN, H), lambda i: (i, 0)),
        out_shape=jax.ShapeDtypeStruct((N, H), jnp.float32),
    )(acc2, x, W_loop_l, b_rel_l, W1_l, b1_l, W2_l, b2_l)


def kernel(nodes_fea, edges, edges_type, W_in, b_in, W_rel, b_rel, W_loop, W1, b1, W2, b2):
    src = edges[0]
    dst = edges[1]
    pad = EP - E
    gidx3 = jnp.concatenate(
        [edges_type * N + src, jnp.zeros((pad,), jnp.int32)]).reshape(NW, NCH, CK)
    dst3 = jnp.concatenate(
        [dst, jnp.full((pad,), N, jnp.int32)]).reshape(NW, NCH, CK)

    x, xr = _initpre(nodes_fea, W_in, b_in.reshape(1, H), W_rel[0])
    for l in range(L):
        acc2 = _sc_agg(xr.reshape(R * N, H), gidx3, dst3)
        args = (acc2, x, W_loop[l], b_rel[l].reshape(1, H), W1[l],
                b1[l].reshape(1, 2 * H), W2[l], b2[l].reshape(1, H))
        if l < L - 1:
            x, xr = _postpre(*args, W_rel[l + 1])
        else:
            x = _post(*args)
    return x
